# Initial kernel scaffold; baseline (speedup 1.0000x reference)
import jax
import jax.numpy as jnp
from jax.experimental import pallas as pl


def kernel(t_normalized, betas):
    raise NotImplementedError("write your pallas kernel here")


# speedup vs baseline: 4.6045x; 4.6045x over previous
import functools

import jax
import jax.numpy as jnp
from jax import lax
from jax.experimental import pallas as pl
from jax.experimental.pallas import tpu as pltpu
from jax.experimental.pallas import tpu_sc as plsc

_TIMESTEPS = 1000
_B = 16384
_NC, _NS, _L = 2, 16, 16
_NW = _NC * _NS
_CHUNK = _B // _NW
_NVEC = _CHUNK // _L
_TBL_PAD = 1024

_mesh = plsc.VectorSubcoreMesh(core_axis_name="c", subcore_axis_name="s")


@functools.partial(
    pl.kernel,
    out_type=jax.ShapeDtypeStruct((_B,), jnp.float32),
    mesh=_mesh,
    scratch_types=[
        pltpu.VMEM((_CHUNK,), jnp.float32),
        pltpu.VMEM((_TBL_PAD,), jnp.float32),
        pltpu.VMEM((_CHUNK,), jnp.float32),
        pltpu.SemaphoreType.DMA,
        pltpu.SemaphoreType.DMA,
    ],
    compiler_params=pltpu.CompilerParams(needs_layout_passes=False),
)
def _betas_lookup(t_hbm, betas_hbm, out_hbm, t_v, tbl_v, out_v, sem_t, sem_b):
    wid = lax.axis_index("s") * _NC + lax.axis_index("c")
    base = wid * _CHUNK
    cp_t = pltpu.async_copy(t_hbm.at[pl.ds(base, _CHUNK)], t_v, sem_t)
    cp_b = pltpu.async_copy(betas_hbm, tbl_v, sem_b)
    cp_t.wait()
    cp_b.wait()
    for i in range(_NVEC):
        x = t_v[pl.ds(i * _L, _L)] * jnp.float32(_TIMESTEPS)
        idx = (x + jnp.float32(0.5)).astype(jnp.int32)
        tie = (idx.astype(jnp.float32) - x) == jnp.float32(0.5)
        odd = (idx & 1) == 1
        idx = jnp.where(tie & odd, idx - 1, idx)
        out_v[pl.ds(i * _L, _L)] = plsc.load_gather(tbl_v, [idx])
    pltpu.sync_copy(out_v, out_hbm.at[pl.ds(base, _CHUNK)])


def kernel(t_normalized, betas):
    betas_padded = jnp.zeros((_TBL_PAD,), jnp.float32).at[: betas.shape[0]].set(betas)
    return _betas_lookup(t_normalized, betas_padded)
